# baseline (speedup 1.0000x reference)
import jax
import jax.numpy as jnp
from jax.experimental import pallas as pl

_KB = 4


def _fused_kernel(g_ref, x_ref, wt_ref, b_ref, o_ref):
    K, N, C = x_ref.shape
    P = wt_ref.shape[1]
    xf = x_ref[...].reshape(K * N, C)
    y = (
        jnp.dot(xf, wt_ref[...], preferred_element_type=jnp.float32)
        + b_ref[...]
    ).reshape(K, N, P)
    for k in range(K):
        o_ref[k] = jnp.maximum(
            jnp.dot(g_ref[k], y[k], preferred_element_type=jnp.float32), 0.0
        )


def kernel(G, x, edge_edge, edge_vert, edge_feat, W, b):
    B, N, C = x.shape
    P = W.shape[0]
    Gm = G.reshape(B, N, N)
    Wt = W.T
    b2 = b.reshape(1, P)
    K = _KB
    out = pl.pallas_call(
        _fused_kernel,
        grid=(B // K,),
        in_specs=[
            pl.BlockSpec((K, N, N), lambda i: (i, 0, 0)),
            pl.BlockSpec((K, N, C), lambda i: (i, 0, 0)),
            pl.BlockSpec((C, P), lambda i: (0, 0)),
            pl.BlockSpec((1, P), lambda i: (0, 0)),
        ],
        out_specs=pl.BlockSpec((K, N, P), lambda i: (i, 0, 0)),
        out_shape=jax.ShapeDtypeStruct((B, N, P), jnp.float32),
    )(Gm, x, Wt, b2)
    return out[None]


# speedup vs baseline: 1.3149x; 1.3149x over previous
import jax
import jax.numpy as jnp
from jax.experimental import pallas as pl

_KB = 16


def _fused_kernel(g_ref, x_ref, wt_ref, b_ref, o_ref):
    K, N, C = x_ref.shape
    P = wt_ref.shape[1]
    xf = x_ref[...].reshape(K * N, C)
    y = (
        jnp.dot(xf, wt_ref[...], preferred_element_type=jnp.float32)
        + b_ref[...]
    ).reshape(K, N, P)
    for k in range(K):
        o_ref[k] = jnp.maximum(
            jnp.dot(g_ref[k], y[k], preferred_element_type=jnp.float32), 0.0
        )


def kernel(G, x, edge_edge, edge_vert, edge_feat, W, b):
    B, N, C = x.shape
    P = W.shape[0]
    Gm = G.reshape(B, N, N)
    Wt = W.T
    b2 = b.reshape(1, P)
    K = _KB
    out = pl.pallas_call(
        _fused_kernel,
        grid=(B // K,),
        in_specs=[
            pl.BlockSpec((K, N, N), lambda i: (i, 0, 0)),
            pl.BlockSpec((K, N, C), lambda i: (i, 0, 0)),
            pl.BlockSpec((C, P), lambda i: (0, 0)),
            pl.BlockSpec((1, P), lambda i: (0, 0)),
        ],
        out_specs=pl.BlockSpec((K, N, P), lambda i: (i, 0, 0)),
        out_shape=jax.ShapeDtypeStruct((B, N, P), jnp.float32),
    )(Gm, x, Wt, b2)
    return out[None]


# speedup vs baseline: 1.3191x; 1.0032x over previous
import jax
import jax.numpy as jnp
from jax.experimental import pallas as pl
from jax.experimental.pallas import tpu as pltpu

_KB = 16


def _fused_kernel(g_ref, x_ref, wt_ref, b_ref, o_ref):
    K, N, C = x_ref.shape
    P = wt_ref.shape[1]
    xf = x_ref[...].reshape(K * N, C)
    y = (
        jnp.dot(xf, wt_ref[...], preferred_element_type=jnp.float32)
        + b_ref[...]
    ).reshape(K, N, P)
    for k in range(K):
        o_ref[k] = jnp.maximum(
            jnp.dot(g_ref[k], y[k], preferred_element_type=jnp.float32), 0.0
        )


def kernel(G, x, edge_edge, edge_vert, edge_feat, W, b):
    B, N, C = x.shape
    P = W.shape[0]
    Gm = G.reshape(B, N, N)
    Wt = W.T
    b2 = b.reshape(1, P)
    K = _KB
    out = pl.pallas_call(
        _fused_kernel,
        grid=(B // K,),
        in_specs=[
            pl.BlockSpec((K, N, N), lambda i: (i, 0, 0)),
            pl.BlockSpec((K, N, C), lambda i: (i, 0, 0)),
            pl.BlockSpec((C, P), lambda i: (0, 0)),
            pl.BlockSpec((1, P), lambda i: (0, 0)),
        ],
        out_specs=pl.BlockSpec((K, N, P), lambda i: (i, 0, 0)),
        out_shape=jax.ShapeDtypeStruct((B, N, P), jnp.float32),
        compiler_params=pltpu.CompilerParams(
            dimension_semantics=("parallel",),
        ),
    )(Gm, x, Wt, b2)
    return out[None]
